# baseline (speedup 1.0000x reference)
import functools

import jax
import jax.numpy as jnp
from jax import lax
from jax.experimental import pallas as pl
from jax.experimental.pallas import tpu as pltpu
from jax.experimental.pallas import tpu_sc as plsc

B = 4096
L = 200
E = 32
N = B * L
NW = 32
PER_W = N // NW
G = 80
GPC = 10
CHUNK = G * GPC
NCHUNK = PER_W // CHUNK
REPS = CHUNK // L
NBUF = 4

_mesh = plsc.VectorSubcoreMesh(core_axis_name="c", subcore_axis_name="s")


@functools.partial(
    pl.kernel,
    mesh=_mesh,
    out_type=jax.ShapeDtypeStruct((N, E), jnp.float32),
    scratch_types=[
        [pltpu.VMEM((CHUNK,), jnp.int32) for _ in range(NBUF)],
        [pltpu.VMEM((CHUNK, E), jnp.float32) for _ in range(NBUF)],
        pltpu.VMEM((L, E), jnp.float32),
        [pltpu.SemaphoreType.DMA for _ in range(NBUF)],
        [pltpu.SemaphoreType.DMA for _ in range(NBUF)],
    ],
    compiler_params=pltpu.CompilerParams(use_tc_tiling_on_sc=False),
)
def _tok_pos_embed(x1d, tok, pos, out, idx_v, buf, pos_v, sem_g, sem_w):
    wid = lax.axis_index("s") * 2 + lax.axis_index("c")
    w_base = wid * PER_W
    pltpu.sync_copy(pos, pos_v)

    def fire(c, b):
        base = w_base + c * CHUNK
        pltpu.sync_copy(x1d.at[pl.ds(base, CHUNK)], idx_v[b])
        for j in range(GPC):
            pltpu.async_copy(
                tok.at[idx_v[b].at[pl.ds(j * G, G)]],
                buf[b].at[pl.ds(j * G, G)],
                sem_g[b],
            )

    def drain_gathers(b):
        for j in range(GPC):
            pltpu.make_async_copy(
                tok.at[idx_v[b].at[pl.ds(j * G, G)]],
                buf[b].at[pl.ds(j * G, G)],
                sem_g[b],
            ).wait()

    def add_pos(b):
        def add_body(l, inner):
            p0 = pos_v[l, pl.ds(0, 16)]
            p1 = pos_v[l, pl.ds(16, 16)]
            for k in range(REPS):
                r = l + L * k
                buf[b][r, pl.ds(0, 16)] += p0
                buf[b][r, pl.ds(16, 16)] += p1
            return inner

        lax.fori_loop(0, L, add_body, 0)

    def write(c, b):
        pltpu.async_copy(buf[b], out.at[pl.ds(w_base + c * CHUNK, CHUNK)], sem_w[b])

    def drain_write(c, b):
        pltpu.make_async_copy(
            buf[b], out.at[pl.ds(w_base + c * CHUNK, CHUNK)], sem_w[b]
        ).wait()

    def slot(c, b, do_fire):
        drain_gathers(b)
        add_pos(b)
        write(c, b)
        if do_fire:
            nb = (b + 1) % NBUF
            if isinstance(c, int):
                if c >= NBUF - 1:
                    drain_write(c - (NBUF - 1), nb)
            else:

                @pl.when(c >= NBUF - 1)
                def _():
                    drain_write(c - (NBUF - 1), nb)

            fire(c + 1, nb)

    fire(0, 0)

    def ring_body(q, carry):
        for bb in range(NBUF):
            slot(q * NBUF + bb, bb, True)
        return carry

    lax.fori_loop(0, (NCHUNK - 1) // NBUF, ring_body, 0)
    for bb in range(NBUF):
        c = (NCHUNK - 1) // NBUF * NBUF + bb
        slot(c, bb, c < NCHUNK - 1)
    for bb in range(NBUF):
        drain_write(NCHUNK - NBUF + bb, bb)


def kernel(x, token_emb, pos_emb):
    x1d = x.reshape(N).astype(jnp.int32)
    out = _tok_pos_embed(x1d, token_emb, pos_emb)
    return out.reshape(B, L, E)
